# baseline (speedup 1.0000x reference)
import functools

import jax
import jax.numpy as jnp
from jax import lax
from jax.experimental import pallas as pl
from jax.experimental.pallas import tpu as pltpu
from jax.experimental.pallas import tpu_sc as plsc

LANES = 16


def _build_sc_kernel(B, D, NC, NS):
    NW = NC * NS
    bpw = B // NW
    mesh = plsc.VectorSubcoreMesh(core_axis_name="c", subcore_axis_name="s")

    @functools.partial(
        pl.kernel,
        mesh=mesh,
        out_type=jax.ShapeDtypeStruct((B,), jnp.float32),
        compiler_params=pltpu.CompilerParams(
            needs_layout_passes=False, use_tc_tiling_on_sc=False),
        scratch_types=[
            pltpu.VMEM((bpw * 2,), jnp.int32),
            pltpu.VMEM((bpw,), jnp.int32),
            pltpu.VMEM((bpw,), jnp.int32),
            pltpu.VMEM((bpw, D), jnp.float32),
            pltpu.VMEM((bpw, D), jnp.float32),
            pltpu.VMEM((bpw,), jnp.float32),
            pltpu.SemaphoreType.DMA,
        ],
    )
    def sc_kernel(x_hbm, wu_hbm, wi_hbm, out_hbm,
                  x_v, uidx_v, iidx_v, urows_v, irows_v, out_v, sem):
        wid = lax.axis_index("s") * NC + lax.axis_index("c")
        base = wid * bpw

        pltpu.sync_copy(x_hbm.at[pl.ds(base * 2, bpw * 2)], x_v)

        lanes = lax.iota(jnp.int32, LANES)

        def split_body(j, carry):
            flat = (j * LANES + lanes) * 2
            uidx_v[pl.ds(j * LANES, LANES)] = plsc.load_gather(x_v, [flat])
            iidx_v[pl.ds(j * LANES, LANES)] = plsc.load_gather(x_v, [flat + 1])
            return carry

        lax.fori_loop(0, bpw // LANES, split_body, 0)

        cu = pltpu.async_copy(wu_hbm.at[uidx_v], urows_v, sem)
        ci = pltpu.async_copy(wi_hbm.at[iidx_v], irows_v, sem)
        cu.wait()
        ci.wait()

        def dot_body(g, carry):
            rows = g * LANES + lanes
            acc = jnp.zeros((LANES,), jnp.float32)
            for d in range(D):
                dcol = jnp.full((LANES,), d, jnp.int32)
                uu = plsc.load_gather(urows_v, [rows, dcol])
                vv = plsc.load_gather(irows_v, [rows, dcol])
                acc = acc + uu * vv
            out_v[pl.ds(g * LANES, LANES)] = acc
            return carry

        lax.fori_loop(0, bpw // LANES, dot_body, 0)

        pltpu.sync_copy(out_v, out_hbm.at[pl.ds(base, bpw)])

    return sc_kernel


def kernel(x, W_user, W_item):
    B = x.shape[0]
    D = W_user.shape[1]
    info = plsc.get_sparse_core_info()
    NC, NS = info.num_cores, info.num_subcores
    sc = _build_sc_kernel(B, D, NC, NS)
    return sc(x.astype(jnp.int32).reshape(B * 2), W_user, W_item)
